# baseline (speedup 1.0000x reference)
import functools

import jax
import jax.numpy as jnp
from jax import lax
from jax.experimental import pallas as pl
from jax.experimental.pallas import tpu as pltpu
from jax.experimental.pallas import tpu_sc as plsc

NCLS = 1000
NROWS = 4096 * 26
NOUT = NROWS * NCLS
ZBLK = 512 * NCLS

NW = 32
ROWS_PER_W = NROWS // NW
LANES = 16
IDX_MINOR = 128
NIDX = ROWS_PER_W // IDX_MINOR

_mesh = plsc.VectorSubcoreMesh(core_axis_name="c", subcore_axis_name="s")


def _zero_block(o_ref):
    o_ref[...] = jnp.zeros((ZBLK,), jnp.float32)


def _tc_zero():
    return pl.pallas_call(
        _zero_block,
        grid=(NOUT // ZBLK,),
        out_specs=pl.BlockSpec((ZBLK,), lambda i: (i,)),
        out_shape=jax.ShapeDtypeStruct((NOUT,), jnp.float32),
    )()


@functools.partial(
    pl.kernel,
    out_type=(),
    mesh=_mesh,
    scratch_types=[
        pltpu.VMEM((ROWS_PER_W,), jnp.int32),
        pltpu.VMEM((NIDX, IDX_MINOR), jnp.int32),
        pltpu.VMEM((IDX_MINOR,), jnp.float32),
        pltpu.SemaphoreType.DMA,
    ],
)
def _sc_scatter(x_hbm, out_ref, idx_v, off_v, ones_v, ssem):
    wid = lax.axis_index("s") * 2 + lax.axis_index("c")
    row0 = wid * ROWS_PER_W
    pltpu.sync_copy(x_hbm.at[pl.ds(row0, ROWS_PER_W)], idx_v)

    for k in range(IDX_MINOR // LANES):
        ones_v[pl.ds(k * LANES, LANES)] = jnp.ones((LANES,), jnp.float32)

    lane_iota = lax.iota(jnp.int32, LANES)

    def offs(j, carry):
        for k in range(IDX_MINOR // LANES):
            ids = idx_v[pl.ds(j * IDX_MINOR + k * LANES, LANES)]
            off_v[j, pl.ds(k * LANES, LANES)] = (
                (lane_iota + row0 + j * IDX_MINOR + k * LANES) * NCLS + ids
            )
        return carry

    lax.fori_loop(0, NIDX, offs, 0)

    def sdma(j, carry):
        pltpu.async_copy(ones_v, out_ref.at[off_v.at[j]], ssem)
        return carry

    lax.fori_loop(0, NIDX, sdma, 0)

    def swait(j, carry):
        pltpu.make_async_copy(
            ones_v, out_ref.at[pl.ds(0, IDX_MINOR)], ssem
        ).wait()
        return carry

    lax.fori_loop(0, NIDX, swait, 0)


def kernel(x):
    xf = x.reshape(-1).astype(jnp.int32)
    zr = jax.new_ref(_tc_zero())
    _sc_scatter(xf, zr)
    return zr[...].reshape(tuple(x.shape) + (NCLS,))
